# Initial kernel scaffold; baseline (speedup 1.0000x reference)
import jax
import jax.numpy as jnp
from jax.experimental import pallas as pl


def kernel(flat, cu_seqlens):
    raise NotImplementedError("write your pallas kernel here")


# speedup vs baseline: 1.2541x; 1.2541x over previous
import functools

import jax
import jax.numpy as jnp
from jax import lax
from jax.experimental import pallas as pl
from jax.experimental.pallas import tpu as pltpu
from jax.experimental.pallas import tpu_sc as plsc

B = 16
MAX_LEN = 2048
D = 1024
HALF = MAX_LEN // 2

NC = 2
NS = 16
NW = NC * NS

C = 32
Z = 32

_mesh = plsc.VectorSubcoreMesh(core_axis_name="c", subcore_axis_name="s")


@functools.partial(
    pl.kernel,
    mesh=_mesh,
    out_type=[
        jax.ShapeDtypeStruct((B * MAX_LEN, D), jnp.float32),
        jax.ShapeDtypeStruct((B, MAX_LEN), jnp.int32),
    ],
    scratch_types=[
        pltpu.VMEM((16,), jnp.int32),
        pltpu.VMEM((16,), jnp.int32),
        pltpu.VMEM((C, D), jnp.float32),
        pltpu.VMEM((Z, D), jnp.float32),
        pltpu.VMEM((HALF,), jnp.int32),
    ],
    compiler_params=pltpu.CompilerParams(use_tc_tiling_on_sc=False,
                                         needs_layout_passes=False),
)
def _pack(starts_hbm, lens_hbm, flat_hbm, padded_hbm, mask_hbm,
          starts_v, lens_v, stage_v, zero_v, mask_v):
    wid = lax.axis_index("s") * NC + lax.axis_index("c")
    b = wid // 2
    p0 = (wid % 2) * HALF

    pltpu.sync_copy(starts_hbm, starts_v)
    pltpu.sync_copy(lens_hbm, lens_v)
    lane = lax.iota(jnp.int32, 16)
    sel = lane == b
    start_b = jnp.sum(jnp.where(sel, starts_v[...], 0))
    len_b = jnp.sum(jnp.where(sel, lens_v[...], 0))

    n_real = jnp.clip(len_b - p0, 0, HALF)
    src0 = start_b + p0
    out0 = b * MAX_LEN + p0

    nf = n_real // C

    def copy_chunk(i, carry):
        pltpu.sync_copy(flat_hbm.at[pl.ds(src0 + i * C, C)], stage_v)
        pltpu.sync_copy(stage_v, padded_hbm.at[pl.ds(out0 + i * C, C)])
        return carry

    lax.fori_loop(0, nf, copy_chunk, 0)

    rem = n_real - nf * C

    @pl.when(jnp.logical_and(rem > 0, n_real >= C))
    def _tail():
        off = n_real - C
        pltpu.sync_copy(flat_hbm.at[pl.ds(src0 + off, C)], stage_v)
        pltpu.sync_copy(stage_v, padded_hbm.at[pl.ds(out0 + off, C)])

    @pl.when(jnp.logical_and(n_real > 0, n_real < C))
    def _tiny():
        def row(i, carry):
            pltpu.sync_copy(flat_hbm.at[pl.ds(src0 + i, 1)],
                            stage_v.at[pl.ds(0, 1)])
            pltpu.sync_copy(stage_v.at[pl.ds(0, 1)],
                            padded_hbm.at[pl.ds(out0 + i, 1)])
            return carry

        lax.fori_loop(0, n_real, row, 0)

    zvec = jnp.zeros((16,), jnp.float32)

    def zfill(r, carry):
        def zcol(c0, carry2):
            zero_v[r, pl.ds(c0 * 16, 16)] = zvec
            return carry2

        lax.fori_loop(0, D // 16, zcol, 0)
        return carry

    lax.fori_loop(0, Z, zfill, 0)

    n_pad = HALF - n_real
    nzf = n_pad // Z

    def zchunk(j, carry):
        pltpu.sync_copy(zero_v,
                        padded_hbm.at[pl.ds(out0 + n_real + j * Z, Z)])
        return carry

    lax.fori_loop(0, nzf, zchunk, 0)

    zrem = n_pad - nzf * Z

    @pl.when(jnp.logical_and(zrem > 0, n_pad >= Z))
    def _ztail():
        pltpu.sync_copy(zero_v, padded_hbm.at[pl.ds(out0 + HALF - Z, Z)])

    @pl.when(jnp.logical_and(n_pad > 0, n_pad < Z))
    def _ztiny():
        def zrow(i, carry):
            pltpu.sync_copy(zero_v.at[pl.ds(0, 1)],
                            padded_hbm.at[pl.ds(out0 + n_real + i, 1)])
            return carry

        lax.fori_loop(0, n_pad, zrow, 0)

    def mrow(k, carry):
        mask_v[pl.ds(k * 16, 16)] = (lane + (p0 + k * 16) < len_b).astype(
            jnp.int32)
        return carry

    lax.fori_loop(0, HALF // 16, mrow, 0)
    pltpu.sync_copy(mask_v, mask_hbm.at[b, pl.ds(p0, HALF)])


def kernel(flat, cu_seqlens):
    starts = cu_seqlens[:B]
    lens = cu_seqlens[1:] - cu_seqlens[:-1]
    padded_flat, mask = _pack(starts, lens, flat)
    return padded_flat.reshape(B, MAX_LEN, D), mask
